# baseline (speedup 1.0000x reference)
import functools

import jax
import jax.numpy as jnp
from jax import lax
from jax.experimental import pallas as pl
from jax.experimental.pallas import tpu as pltpu
from jax.experimental.pallas import tpu_sc as plsc

NC, NS = 2, 16
NW = NC * NS
B = 16384
D = 64
CHUNK = 128
IDX_ROWS = B // CHUNK

BLK = 8192


@functools.cache
def _make_sc_gather(nstream):
    mesh = plsc.VectorSubcoreMesh(
        core_axis_name="c", subcore_axis_name="s", num_cores=NC, num_subcores=NS
    )
    rows_per_w = B // NW
    nchunk = rows_per_w // CHUNK

    @functools.partial(
        pl.kernel,
        mesh=mesh,
        out_type=tuple(
            jax.ShapeDtypeStruct((B, D), jnp.float32) for _ in range(nstream)
        ),
        scratch_types=(
            pltpu.VMEM((nchunk, CHUNK), jnp.int32),
            pltpu.VMEM((rows_per_w, D), jnp.float32),
            pltpu.SemaphoreType.DMA,
        ),
        compiler_params=pltpu.CompilerParams(use_tc_tiling_on_sc=True),
    )
    def _sc_gather(tab, *args):
        idxs = args[:nstream]
        outs = args[nstream:2 * nstream]
        idx_v, rows_v, sem = args[2 * nstream:]
        wid = lax.axis_index("s") * NC + lax.axis_index("c")
        row0 = wid * nchunk
        for s in range(nstream):
            pltpu.sync_copy(idxs[s].at[pl.ds(row0, nchunk)], idx_v)

            def issue(g, _):
                c = g // (CHUNK // 16)
                off = (g - c * (CHUNK // 16)) * 16
                vec = idx_v[c, pl.ds(off, 16)]
                for k in range(16):
                    i = vec[k]
                    pltpu.async_copy(tab.at[pl.ds(i, 1)],
                                     rows_v.at[pl.ds(g * 16 + k, 1)], sem)
                return _

            lax.fori_loop(0, rows_per_w // 16, issue, 0)
            pltpu.make_async_copy(tab.at[pl.ds(0, rows_per_w)], rows_v,
                                  sem).wait()
            pltpu.sync_copy(rows_v, outs[s].at[pl.ds(row0 * CHUNK, rows_per_w)])

    return _sc_gather


def _tc_body(u_ref, p_ref, n_ref, w0_ref, b0_ref, w1_ref, b1_ref,
             w2_ref, b2_ref, o_ref):
    u = u_ref[...]
    p = p_ref[...]
    n = n_ref[...]
    acc = jnp.sum(u * (p - n), axis=1, keepdims=True)
    x = jnp.concatenate([u, p, n], axis=0).astype(jnp.bfloat16)
    for w_ref, b_ref in ((w0_ref, b0_ref), (w1_ref, b1_ref), (w2_ref, b2_ref)):
        w = w_ref[...].astype(jnp.bfloat16)
        b = b_ref[...]
        y = jnp.dot(x, w, preferred_element_type=jnp.float32) + b
        yf = jnp.maximum(y, 0.3 * y)
        x = yf.astype(jnp.bfloat16)
        uf = yf[:BLK]
        pf = yf[BLK:2 * BLK]
        nf = yf[2 * BLK:]
        acc = acc + jnp.sum(uf * (pf - nf), axis=1, keepdims=True)
    o_ref[...] = acc.reshape(BLK // CHUNK, CHUNK)


_tc_score = pl.pallas_call(
    _tc_body,
    grid=(B // BLK,),
    in_specs=[
        pl.BlockSpec((BLK, D), lambda i: (i, 0)),
        pl.BlockSpec((BLK, D), lambda i: (i, 0)),
        pl.BlockSpec((BLK, D), lambda i: (i, 0)),
        pl.BlockSpec((D, D), lambda i: (0, 0)),
        pl.BlockSpec((1, D), lambda i: (0, 0)),
        pl.BlockSpec((D, D), lambda i: (0, 0)),
        pl.BlockSpec((1, D), lambda i: (0, 0)),
        pl.BlockSpec((D, D), lambda i: (0, 0)),
        pl.BlockSpec((1, D), lambda i: (0, 0)),
    ],
    out_specs=pl.BlockSpec((BLK // CHUNK, CHUNK), lambda i: (i, 0)),
    out_shape=jax.ShapeDtypeStruct((B // CHUNK, CHUNK), jnp.float32),
)


def kernel(user_indices, pos_item_indices, neg_item_indices, user_table,
           item_table, W1_0, b1_0, W1_1, b1_1, W1_2, b1_2):
    ui = user_indices.astype(jnp.int32).reshape(IDX_ROWS, CHUNK)
    pi = pos_item_indices.astype(jnp.int32).reshape(IDX_ROWS, CHUNK)
    ni = neg_item_indices.astype(jnp.int32).reshape(IDX_ROWS, CHUNK)
    (u,) = _make_sc_gather(1)(user_table, ui)
    p, n = _make_sc_gather(2)(item_table, pi, ni)
    out = _tc_score(u, p, n,
                    W1_0, b1_0.reshape(1, D),
                    W1_1, b1_1.reshape(1, D),
                    W1_2, b1_2.reshape(1, D))
    return out.reshape(B, 1)


# speedup vs baseline: 1.0206x; 1.0206x over previous
import functools

import jax
import jax.numpy as jnp
from jax import lax
from jax.experimental import pallas as pl
from jax.experimental.pallas import tpu as pltpu
from jax.experimental.pallas import tpu_sc as plsc

NC, NS = 2, 16
NW = NC * NS
B = 16384
D = 64
CHUNK = 128
IDX_ROWS = B // CHUNK

BLK = 2048


@functools.cache
def _make_sc_gather(nstream):
    mesh = plsc.VectorSubcoreMesh(
        core_axis_name="c", subcore_axis_name="s", num_cores=NC, num_subcores=NS
    )
    rows_per_w = B // NW
    nchunk = rows_per_w // CHUNK

    @functools.partial(
        pl.kernel,
        mesh=mesh,
        out_type=tuple(
            jax.ShapeDtypeStruct((B, D), jnp.float32) for _ in range(nstream)
        ),
        scratch_types=(
            pltpu.VMEM((nchunk, CHUNK), jnp.int32),
            pltpu.VMEM((rows_per_w, D), jnp.float32),
            pltpu.SemaphoreType.DMA,
        ),
        compiler_params=pltpu.CompilerParams(use_tc_tiling_on_sc=True),
    )
    def _sc_gather(tab, *args):
        idxs = args[:nstream]
        outs = args[nstream:2 * nstream]
        idx_v, rows_v, sem = args[2 * nstream:]
        wid = lax.axis_index("s") * NC + lax.axis_index("c")
        row0 = wid * nchunk
        for s in range(nstream):
            pltpu.sync_copy(idxs[s].at[pl.ds(row0, nchunk)], idx_v)

            def issue(g, _):
                c = g // (CHUNK // 16)
                off = (g - c * (CHUNK // 16)) * 16
                vec = idx_v[c, pl.ds(off, 16)]
                for k in range(16):
                    i = vec[k]
                    pltpu.async_copy(tab.at[pl.ds(i, 1)],
                                     rows_v.at[pl.ds(g * 16 + k, 1)], sem)
                return _

            lax.fori_loop(0, rows_per_w // 16, issue, 0)
            pltpu.make_async_copy(tab.at[pl.ds(0, rows_per_w)], rows_v,
                                  sem).wait()
            pltpu.sync_copy(rows_v, outs[s].at[pl.ds(row0 * CHUNK, rows_per_w)])

    return _sc_gather


def _tc_body(u_ref, p_ref, n_ref, w0_ref, b0_ref, w1_ref, b1_ref,
             w2_ref, b2_ref, o_ref):
    u = u_ref[...]
    p = p_ref[...]
    n = n_ref[...]
    acc = jnp.sum(u * (p - n), axis=1, keepdims=True)
    x = jnp.concatenate([u, p, n], axis=0).astype(jnp.bfloat16)
    for w_ref, b_ref in ((w0_ref, b0_ref), (w1_ref, b1_ref), (w2_ref, b2_ref)):
        w = w_ref[...].astype(jnp.bfloat16)
        b = b_ref[...]
        y = jnp.dot(x, w, preferred_element_type=jnp.float32) + b
        yf = jnp.maximum(y, 0.3 * y)
        x = yf.astype(jnp.bfloat16)
        uf = yf[:BLK]
        pf = yf[BLK:2 * BLK]
        nf = yf[2 * BLK:]
        acc = acc + jnp.sum(uf * (pf - nf), axis=1, keepdims=True)
    o_ref[...] = acc.reshape(BLK // CHUNK, CHUNK)


_tc_score = pl.pallas_call(
    _tc_body,
    grid=(B // BLK,),
    in_specs=[
        pl.BlockSpec((BLK, D), lambda i: (i, 0)),
        pl.BlockSpec((BLK, D), lambda i: (i, 0)),
        pl.BlockSpec((BLK, D), lambda i: (i, 0)),
        pl.BlockSpec((D, D), lambda i: (0, 0)),
        pl.BlockSpec((1, D), lambda i: (0, 0)),
        pl.BlockSpec((D, D), lambda i: (0, 0)),
        pl.BlockSpec((1, D), lambda i: (0, 0)),
        pl.BlockSpec((D, D), lambda i: (0, 0)),
        pl.BlockSpec((1, D), lambda i: (0, 0)),
    ],
    out_specs=pl.BlockSpec((BLK // CHUNK, CHUNK), lambda i: (i, 0)),
    out_shape=jax.ShapeDtypeStruct((B // CHUNK, CHUNK), jnp.float32),
)


def kernel(user_indices, pos_item_indices, neg_item_indices, user_table,
           item_table, W1_0, b1_0, W1_1, b1_1, W1_2, b1_2):
    ui = user_indices.astype(jnp.int32).reshape(IDX_ROWS, CHUNK)
    pi = pos_item_indices.astype(jnp.int32).reshape(IDX_ROWS, CHUNK)
    ni = neg_item_indices.astype(jnp.int32).reshape(IDX_ROWS, CHUNK)
    (u,) = _make_sc_gather(1)(user_table, ui)
    p, n = _make_sc_gather(2)(item_table, pi, ni)
    out = _tc_score(u, p, n,
                    W1_0, b1_0.reshape(1, D),
                    W1_1, b1_1.reshape(1, D),
                    W1_2, b1_2.reshape(1, D))
    return out.reshape(B, 1)
